# Initial kernel scaffold; baseline (speedup 1.0000x reference)
import jax
import jax.numpy as jnp
from jax.experimental import pallas as pl


def kernel(h_ijk, g, edge_type, Wr, br, W1, b1):
    raise NotImplementedError("write your pallas kernel here")


# speedup vs baseline: 2.2175x; 2.2175x over previous
import jax
import jax.numpy as jnp
from jax.experimental import pallas as pl
from jax.experimental.pallas import tpu as pltpu

E = 320000
D = 128
R = 1000
RPAD = 1024
BLK = 512
NBLK = E // BLK


def _seg_kernel(et_ref, h_ref, g_ref, wr_ref, w1_ref, br_ref, b1_ref,
                out_ref, acc_ref):
    i = pl.program_id(0)

    @pl.when(i == 0)
    def _():
        acc_ref[...] = jnp.zeros_like(acc_ref)

    et = et_ref[0, 0, :]
    rows = jax.lax.broadcasted_iota(jnp.int32, (RPAD, BLK), 0)
    onehot_t = (rows == et[None, :]).astype(jnp.float32)
    acc_ref[...] += jnp.dot(onehot_t, h_ref[...],
                            preferred_element_type=jnp.float32)

    @pl.when(i == NBLK - 1)
    def _():
        g_edges = acc_ref[:R, :]
        norm = jnp.sqrt(jnp.sum(g_edges * g_edges, axis=1, keepdims=True))
        g_edges = g_edges / jnp.maximum(norm, 1e-12)
        g_edges = jnp.where(g_edges > 0, g_edges, jnp.exp(g_edges) - 1.0)
        t1 = jax.lax.dot_general(g_ref[...], wr_ref[...],
                                 (((1,), (1,)), ((), ())),
                                 preferred_element_type=jnp.float32)
        t2 = jax.lax.dot_general(g_edges, w1_ref[...],
                                 (((1,), (1,)), ((), ())),
                                 preferred_element_type=jnp.float32)
        out_ref[...] = t1 + t2 + br_ref[...][None, :] + b1_ref[...][None, :]


def kernel(h_ijk, g, edge_type, Wr, br, W1, b1):
    et = jnp.asarray(edge_type, jnp.int32).reshape(NBLK, 1, BLK)
    return pl.pallas_call(
        _seg_kernel,
        grid=(NBLK,),
        in_specs=[
            pl.BlockSpec((1, 1, BLK), lambda i: (i, 0, 0)),
            pl.BlockSpec((BLK, D), lambda i: (i, 0)),
            pl.BlockSpec(g.shape, lambda i: (0, 0)),
            pl.BlockSpec(Wr.shape, lambda i: (0, 0)),
            pl.BlockSpec(W1.shape, lambda i: (0, 0)),
            pl.BlockSpec(br.shape, lambda i: (0,)),
            pl.BlockSpec(b1.shape, lambda i: (0,)),
        ],
        out_specs=pl.BlockSpec((R, 64), lambda i: (0, 0)),
        out_shape=jax.ShapeDtypeStruct((R, 64), jnp.float32),
        scratch_shapes=[pltpu.VMEM((RPAD, D), jnp.float32)],
    )(et, h_ijk, g, Wr, W1, br, b1)
